# baseline (speedup 1.0000x reference)
import functools

import jax
import jax.numpy as jnp
from jax import lax
from jax.experimental import pallas as pl
from jax.experimental.pallas import tpu as pltpu
from jax.experimental.pallas import tpu_sc as plsc

N = 16384
C = 1000
CPAD = 1024

NC = 2
NS = 16
NW = NC * NS
CHUNK = N // NW
JROWS = CHUNK // 128

ROWS_PER_BLOCK = 40
GRID = C // ROWS_PER_BLOCK


def _ce_body(x_ref, lab_ref, out_ref, macc, sacc, pacc):
    i = pl.program_id(0)

    @pl.when(i == 0)
    def _init():
        macc[...] = jnp.full((1, N), -jnp.inf, jnp.float32)
        sacc[...] = jnp.zeros((1, N), jnp.float32)
        pacc[...] = jnp.zeros((1, N), jnp.float32)

    x = x_ref[...]
    lab = lab_ref[...]
    mold = macc[...]
    bm = jnp.max(x, axis=0, keepdims=True)
    mnew = jnp.maximum(mold, bm)
    e = jnp.exp(x - mnew)
    cls = lax.broadcasted_iota(jnp.int32, x.shape, 0) + i * ROWS_PER_BLOCK
    px = jnp.where(cls == lab[None, :], x, 0.0)
    ones_row = jnp.ones((1, ROWS_PER_BLOCK), jnp.float32)
    s_b = lax.dot_general(ones_row, e, (((1,), (0,)), ((), ())),
                          preferred_element_type=jnp.float32)
    p_b = lax.dot_general(ones_row, px, (((1,), (0,)), ((), ())),
                          preferred_element_type=jnp.float32)
    sacc[...] = sacc[...] * jnp.exp(mold - mnew) + s_b
    macc[...] = mnew
    pacc[...] = pacc[...] + p_b

    @pl.when(i == GRID - 1)
    def _finish():
        out_ref[...] = (macc[...] + jnp.log(sacc[...]) - pacc[...]
                        ).reshape(N)


def _losses_tc(inputs_t, labels):
    return pl.pallas_call(
        _ce_body,
        grid=(GRID,),
        in_specs=[
            pl.BlockSpec((ROWS_PER_BLOCK, N), lambda i: (i, 0)),
            pl.BlockSpec((N,), lambda i: (0,)),
        ],
        out_specs=pl.BlockSpec((N,), lambda i: (0,)),
        out_shape=jax.ShapeDtypeStruct((N,), jnp.float32),
        scratch_shapes=[
            pltpu.VMEM((1, N), jnp.float32),
            pltpu.VMEM((1, N), jnp.float32),
            pltpu.VMEM((1, N), jnp.float32),
        ],
        compiler_params=pltpu.CompilerParams(
            dimension_semantics=("arbitrary",),
        ),
    )(inputs_t, labels)


def _sc_segment_sums(losses_r, labels_r):
    mesh = plsc.VectorSubcoreMesh(
        core_axis_name="c", subcore_axis_name="s",
        num_cores=NC, num_subcores=NS,
    )

    @functools.partial(
        pl.kernel,
        mesh=mesh,
        out_type=jax.ShapeDtypeStruct((NC, 2 * CPAD), jnp.float32),
        scratch_types=[
            pltpu.VMEM((JROWS, 128), jnp.int32),
            pltpu.VMEM((JROWS, 128), jnp.float32),
            pltpu.VMEM((128,), jnp.float32),
            pltpu.VMEM((CPAD,), jnp.float32),
            pltpu.VMEM_SHARED((CPAD,), jnp.float32),
            pltpu.VMEM_SHARED((CPAD,), jnp.float32),
        ],
    )
    def k(losses_hbm, labels_hbm, part_out,
          labv, lossv, onesv, zv, sh_sum, sh_cnt):
        c = lax.axis_index("c")
        s = lax.axis_index("s")
        wid = c * NS + s

        one16 = jnp.ones((16,), jnp.float32)
        for i in range(128 // 16):
            onesv[pl.ds(i * 16, 16)] = one16

        @pl.when(s == 0)
        def _zero():
            z16 = jnp.zeros((16,), jnp.float32)
            for i in range(CPAD // 16):
                zv[pl.ds(i * 16, 16)] = z16
            pltpu.sync_copy(zv, sh_sum)
            pltpu.sync_copy(zv, sh_cnt)

        pltpu.sync_copy(labels_hbm.at[wid], labv)
        pltpu.sync_copy(losses_hbm.at[wid], lossv)

        plsc.subcore_barrier()

        for j in range(JROWS):
            pltpu.sync_copy(lossv.at[j], sh_sum.at[labv.at[j]], add=True)
            pltpu.sync_copy(onesv, sh_cnt.at[labv.at[j]], add=True)

        plsc.subcore_barrier()

        @pl.when(s == 0)
        def _writeout():
            pltpu.sync_copy(sh_sum, part_out.at[c, pl.ds(0, CPAD)])
            pltpu.sync_copy(sh_cnt, part_out.at[c, pl.ds(CPAD, CPAD)])

    return k(losses_r, labels_r)


def kernel(inputs, labels):
    losses = _losses_tc(inputs.T, labels)
    losses_r = losses.reshape(NW, JROWS, 128)
    labels_r = labels.reshape(NW, JROWS, 128)
    part = _sc_segment_sums(losses_r, labels_r)
    tot = part.sum(axis=0)
    return (tot[:C], tot[CPAD:CPAD + C])


# speedup vs baseline: 1.2056x; 1.2056x over previous
import functools

import jax
import jax.numpy as jnp
from jax import lax
from jax.experimental import pallas as pl
from jax.experimental.pallas import tpu as pltpu
from jax.experimental.pallas import tpu_sc as plsc

N = 16384
C = 1000
CPAD = 1024

NC = 2
NS = 16
NW = NC * NS
CHUNK = N // NW
JROWS = CHUNK // 128

ROWS_PER_BLOCK = 200
GRID = C // ROWS_PER_BLOCK
COL_SPLIT = 2
NCOL = N // COL_SPLIT


def _ce_body(x_ref, lab_ref, out_ref, macc, sacc, pacc):
    i = pl.program_id(1)

    @pl.when(i == 0)
    def _init():
        macc[...] = jnp.full((1, NCOL), -jnp.inf, jnp.float32)
        sacc[...] = jnp.zeros((1, NCOL), jnp.float32)
        pacc[...] = jnp.zeros((1, NCOL), jnp.float32)

    x = x_ref[...]
    lab = lab_ref[...]
    mold = macc[...]
    bm = jnp.max(x, axis=0, keepdims=True)
    mnew = jnp.maximum(mold, bm)
    e = jnp.exp(x - mnew)
    cls = lax.broadcasted_iota(jnp.int32, x.shape, 0) + i * ROWS_PER_BLOCK
    px = jnp.where(cls == lab[None, :], x, 0.0)
    ones_row = jnp.ones((1, ROWS_PER_BLOCK), jnp.float32)
    s_b = lax.dot_general(ones_row, e, (((1,), (0,)), ((), ())),
                          preferred_element_type=jnp.float32)
    p_b = lax.dot_general(ones_row, px, (((1,), (0,)), ((), ())),
                          preferred_element_type=jnp.float32)
    sacc[...] = sacc[...] * jnp.exp(mold - mnew) + s_b
    macc[...] = mnew
    pacc[...] = pacc[...] + p_b

    @pl.when(i == GRID - 1)
    def _finish():
        out_ref[...] = (macc[...] + jnp.log(sacc[...]) - pacc[...]
                        ).reshape(NCOL)


def _losses_tc(inputs_t, labels):
    return pl.pallas_call(
        _ce_body,
        grid=(COL_SPLIT, GRID),
        in_specs=[
            pl.BlockSpec((ROWS_PER_BLOCK, NCOL), lambda j, i: (i, j)),
            pl.BlockSpec((NCOL,), lambda j, i: (j,)),
        ],
        out_specs=pl.BlockSpec((NCOL,), lambda j, i: (j,)),
        out_shape=jax.ShapeDtypeStruct((N,), jnp.float32),
        scratch_shapes=[
            pltpu.VMEM((1, NCOL), jnp.float32),
            pltpu.VMEM((1, NCOL), jnp.float32),
            pltpu.VMEM((1, NCOL), jnp.float32),
        ],
        compiler_params=pltpu.CompilerParams(
            dimension_semantics=("arbitrary", "arbitrary"),
        ),
    )(inputs_t, labels)


def _sc_segment_sums(losses_r, labels_r):
    mesh = plsc.VectorSubcoreMesh(
        core_axis_name="c", subcore_axis_name="s",
        num_cores=NC, num_subcores=NS,
    )

    @functools.partial(
        pl.kernel,
        mesh=mesh,
        out_type=jax.ShapeDtypeStruct((NC, 2 * CPAD), jnp.float32),
        scratch_types=[
            pltpu.VMEM((JROWS, 128), jnp.int32),
            pltpu.VMEM((JROWS, 128), jnp.float32),
            pltpu.VMEM((128,), jnp.float32),
            pltpu.VMEM((CPAD,), jnp.float32),
            pltpu.VMEM_SHARED((CPAD,), jnp.float32),
            pltpu.VMEM_SHARED((CPAD,), jnp.float32),
        ],
    )
    def k(losses_hbm, labels_hbm, part_out,
          labv, lossv, onesv, zv, sh_sum, sh_cnt):
        c = lax.axis_index("c")
        s = lax.axis_index("s")
        wid = c * NS + s

        one16 = jnp.ones((16,), jnp.float32)
        for i in range(128 // 16):
            onesv[pl.ds(i * 16, 16)] = one16

        @pl.when(s == 0)
        def _zero():
            z16 = jnp.zeros((16,), jnp.float32)
            for i in range(CPAD // 16):
                zv[pl.ds(i * 16, 16)] = z16
            pltpu.sync_copy(zv, sh_sum)
            pltpu.sync_copy(zv, sh_cnt)

        pltpu.sync_copy(labels_hbm.at[wid], labv)
        pltpu.sync_copy(losses_hbm.at[wid], lossv)

        plsc.subcore_barrier()

        for j in range(JROWS):
            pltpu.sync_copy(lossv.at[j], sh_sum.at[labv.at[j]], add=True)
            pltpu.sync_copy(onesv, sh_cnt.at[labv.at[j]], add=True)

        plsc.subcore_barrier()

        @pl.when(s == 0)
        def _writeout():
            pltpu.sync_copy(sh_sum, part_out.at[c, pl.ds(0, CPAD)])
            pltpu.sync_copy(sh_cnt, part_out.at[c, pl.ds(CPAD, CPAD)])

    return k(losses_r, labels_r)


def kernel(inputs, labels):
    losses = _losses_tc(inputs.T, labels)
    losses_r = losses.reshape(NW, JROWS, 128)
    labels_r = labels.reshape(NW, JROWS, 128)
    part = _sc_segment_sums(losses_r, labels_r)
    tot = part.sum(axis=0)
    return (tot[:C], tot[CPAD:CPAD + C])


# speedup vs baseline: 1.2401x; 1.0286x over previous
import functools

import jax
import jax.numpy as jnp
from jax import lax
from jax.experimental import pallas as pl
from jax.experimental.pallas import tpu as pltpu
from jax.experimental.pallas import tpu_sc as plsc

N = 16384
C = 1000
CPAD = 1024

NC = 2
NS = 16
NW = NC * NS
CHUNK = N // NW
JROWS = CHUNK // 128

ROWS_PER_BLOCK = 200
GRID = C // ROWS_PER_BLOCK
COL_SPLIT = 1
NCOL = N // COL_SPLIT


def _ce_body(x_ref, lab_ref, out_ref, macc, sacc, pacc):
    i = pl.program_id(1)

    @pl.when(i == 0)
    def _init():
        macc[...] = jnp.full((1, NCOL), -jnp.inf, jnp.float32)
        sacc[...] = jnp.zeros((1, NCOL), jnp.float32)
        pacc[...] = jnp.zeros((1, NCOL), jnp.float32)

    x = x_ref[...]
    lab = lab_ref[...]
    mold = macc[...]
    bm = jnp.max(x, axis=0, keepdims=True)
    mnew = jnp.maximum(mold, bm)
    e = jnp.exp(x - mnew)
    cls = lax.broadcasted_iota(jnp.int32, x.shape, 0) + i * ROWS_PER_BLOCK
    px = jnp.where(cls == lab[None, :], x, 0.0)
    ones_row = jnp.ones((1, ROWS_PER_BLOCK), jnp.float32)
    s_b = lax.dot_general(ones_row, e, (((1,), (0,)), ((), ())),
                          preferred_element_type=jnp.float32)
    p_b = lax.dot_general(ones_row, px, (((1,), (0,)), ((), ())),
                          preferred_element_type=jnp.float32)
    sacc[...] = sacc[...] * jnp.exp(mold - mnew) + s_b
    macc[...] = mnew
    pacc[...] = pacc[...] + p_b

    @pl.when(i == GRID - 1)
    def _finish():
        out_ref[...] = (macc[...] + jnp.log(sacc[...]) - pacc[...]
                        ).reshape(NCOL)


def _losses_tc(inputs_t, labels):
    return pl.pallas_call(
        _ce_body,
        grid=(COL_SPLIT, GRID),
        in_specs=[
            pl.BlockSpec((ROWS_PER_BLOCK, NCOL), lambda j, i: (i, j)),
            pl.BlockSpec((NCOL,), lambda j, i: (j,)),
        ],
        out_specs=pl.BlockSpec((NCOL,), lambda j, i: (j,)),
        out_shape=jax.ShapeDtypeStruct((N,), jnp.float32),
        scratch_shapes=[
            pltpu.VMEM((1, NCOL), jnp.float32),
            pltpu.VMEM((1, NCOL), jnp.float32),
            pltpu.VMEM((1, NCOL), jnp.float32),
        ],
        compiler_params=pltpu.CompilerParams(
            dimension_semantics=("arbitrary", "arbitrary"),
        ),
    )(inputs_t, labels)


def _sc_segment_sums(losses_r, labels_r):
    mesh = plsc.VectorSubcoreMesh(
        core_axis_name="c", subcore_axis_name="s",
        num_cores=NC, num_subcores=NS,
    )

    @functools.partial(
        pl.kernel,
        mesh=mesh,
        out_type=jax.ShapeDtypeStruct((NC, 2 * CPAD), jnp.float32),
        scratch_types=[
            pltpu.VMEM((JROWS, 128), jnp.int32),
            pltpu.VMEM((JROWS, 128), jnp.float32),
            pltpu.VMEM((128,), jnp.float32),
            pltpu.VMEM((CPAD,), jnp.float32),
            pltpu.VMEM_SHARED((CPAD,), jnp.float32),
            pltpu.VMEM_SHARED((CPAD,), jnp.float32),
        ],
    )
    def k(losses_hbm, labels_hbm, part_out,
          labv, lossv, onesv, zv, sh_sum, sh_cnt):
        c = lax.axis_index("c")
        s = lax.axis_index("s")
        wid = c * NS + s

        one16 = jnp.ones((16,), jnp.float32)
        for i in range(128 // 16):
            onesv[pl.ds(i * 16, 16)] = one16

        @pl.when(s == 0)
        def _zero():
            z16 = jnp.zeros((16,), jnp.float32)
            for i in range(CPAD // 16):
                zv[pl.ds(i * 16, 16)] = z16
            pltpu.sync_copy(zv, sh_sum)
            pltpu.sync_copy(zv, sh_cnt)

        pltpu.sync_copy(labels_hbm.at[wid], labv)
        pltpu.sync_copy(losses_hbm.at[wid], lossv)

        plsc.subcore_barrier()

        for j in range(JROWS):
            pltpu.sync_copy(lossv.at[j], sh_sum.at[labv.at[j]], add=True)
            pltpu.sync_copy(onesv, sh_cnt.at[labv.at[j]], add=True)

        plsc.subcore_barrier()

        @pl.when(s == 0)
        def _writeout():
            pltpu.sync_copy(sh_sum, part_out.at[c, pl.ds(0, CPAD)])
            pltpu.sync_copy(sh_cnt, part_out.at[c, pl.ds(CPAD, CPAD)])

    return k(losses_r, labels_r)


def kernel(inputs, labels):
    losses = _losses_tc(inputs.T, labels)
    losses_r = losses.reshape(NW, JROWS, 128)
    labels_r = labels.reshape(NW, JROWS, 128)
    part = _sc_segment_sums(losses_r, labels_r)
    tot = part.sum(axis=0)
    return (tot[:C], tot[CPAD:CPAD + C])
